# Initial kernel scaffold; baseline (speedup 1.0000x reference)
import jax
import jax.numpy as jnp
from jax.experimental import pallas as pl


def kernel(driverID, weekID, timeID, dist, W_driver, W_week, W_time):
    raise NotImplementedError("write your pallas kernel here")


# speedup vs baseline: 3.0702x; 3.0702x over previous
import functools

import jax
import jax.numpy as jnp
from jax import lax
from jax.experimental import pallas as pl
from jax.experimental.pallas import tpu as pltpu
from jax.experimental.pallas import tpu_sc as plsc

N = 16384
D_DRV, D_WK, D_TM = 16, 3, 8
D_OUT = D_DRV + D_WK + D_TM + 1
WK_STRIDE = 4

_info = plsc.get_sparse_core_info()
NC, NS, L = _info.num_cores, _info.num_subcores, _info.num_lanes
NW = NC * NS
B_W = N // NW
CHUNKS = B_W // L


def _body(drv_idx_hbm, wk_idx_hbm, tm_idx_hbm, dist_hbm,
          wd_hbm, wk_hbm, wt_hbm, out_hbm,
          drv_idx_v, wk_idx_v, tm_idx_v, dist_v,
          drv_rows_v, wk_tab_v, tm_tab_v, out_v, sem):
    wid = lax.axis_index("s") * NC + lax.axis_index("c")
    base = wid * B_W

    pltpu.sync_copy(drv_idx_hbm.at[pl.ds(base, B_W)], drv_idx_v)
    gather = pltpu.async_copy(wd_hbm.at[drv_idx_v], drv_rows_v, sem)

    pltpu.sync_copy(wk_idx_hbm.at[pl.ds(base, B_W)], wk_idx_v)
    pltpu.sync_copy(tm_idx_hbm.at[pl.ds(base, B_W)], tm_idx_v)
    pltpu.sync_copy(dist_hbm.at[pl.ds(base, B_W)], dist_v)
    pltpu.sync_copy(wk_hbm, wk_tab_v)
    pltpu.sync_copy(wt_hbm, tm_tab_v)
    gather.wait()

    iota = lax.iota(jnp.int32, L)

    def chunk(i, carry):
        r = i * L
        rows = r + iota
        obase = rows * D_OUT
        for k in range(L):
            row_v = drv_rows_v[r + k]
            plsc.store_scatter(out_v, [(r + k) * D_OUT + iota], row_v)
        wk16 = plsc.load_gather(wk_idx_v, [rows]) * WK_STRIDE
        for c in range(D_WK):
            val = plsc.load_gather(wk_tab_v, [wk16 + c])
            plsc.store_scatter(out_v, [obase + (D_DRV + c)], val)
        tm16 = plsc.load_gather(tm_idx_v, [rows]) * D_TM
        for c in range(D_TM):
            val = plsc.load_gather(tm_tab_v, [tm16 + c])
            plsc.store_scatter(out_v, [obase + (D_DRV + D_WK + c)], val)
        d16 = plsc.load_gather(dist_v, [rows])
        plsc.store_scatter(out_v, [obase + (D_OUT - 1)], d16)
        return carry

    lax.fori_loop(0, CHUNKS, chunk, 0)

    pltpu.sync_copy(out_v, out_hbm.at[pl.ds(base * D_OUT, B_W * D_OUT)])


@jax.jit
def _run(drv_idx, wk_idx, tm_idx, dist, wd, wk_flat, wt_flat):
    mesh = plsc.VectorSubcoreMesh(core_axis_name="c", subcore_axis_name="s")
    f = pl.kernel(
        _body, mesh=mesh,
        compiler_params=pltpu.CompilerParams(
            needs_layout_passes=False, use_tc_tiling_on_sc=False),
        out_type=jax.ShapeDtypeStruct((N * D_OUT,), jnp.float32),
        scratch_types=[
            pltpu.VMEM((B_W,), jnp.int32),
            pltpu.VMEM((B_W,), jnp.int32),
            pltpu.VMEM((B_W,), jnp.int32),
            pltpu.VMEM((B_W,), jnp.float32),
            pltpu.VMEM((B_W, D_DRV), jnp.float32),
            pltpu.VMEM((8 * WK_STRIDE,), jnp.float32),
            pltpu.VMEM((1440 * D_TM,), jnp.float32),
            pltpu.VMEM((B_W * D_OUT,), jnp.float32),
            pltpu.SemaphoreType.DMA,
        ],
    )
    return f(drv_idx, wk_idx, tm_idx, dist, wd, wk_flat, wt_flat)


def kernel(driverID, weekID, timeID, dist, W_driver, W_week, W_time):
    drv_idx = driverID.astype(jnp.int32).reshape(-1)
    wk_idx = weekID.astype(jnp.int32).reshape(-1)
    tm_idx = timeID.astype(jnp.int32).reshape(-1)
    wk_pad = jnp.zeros((8, WK_STRIDE), jnp.float32).at[:7, :D_WK].set(W_week)
    out_flat = _run(drv_idx, wk_idx, tm_idx, dist.reshape(-1),
                    W_driver, wk_pad.reshape(-1), W_time.reshape(-1))
    return out_flat.reshape(N, D_OUT)


# speedup vs baseline: 3.4523x; 1.1245x over previous
import jax
import jax.numpy as jnp
from jax import lax
from jax.experimental import pallas as pl
from jax.experimental.pallas import tpu as pltpu
from jax.experimental.pallas import tpu_sc as plsc

N = 16384
D_DRV, D_WK, D_TM = 16, 3, 8
D_OUT = D_DRV + D_WK + D_TM + 1

_info = plsc.get_sparse_core_info()
NC, NS, L = _info.num_cores, _info.num_subcores, _info.num_lanes
NW = NC * NS
B_W = N // NW


D_REST = D_OUT - D_DRV
WK_STRIDE = 4
CHUNKS = B_W // L


def _body(drv_idx_hbm, wk_idx_hbm, tm_idx_hbm, dist_hbm,
          wd_hbm, wk_hbm, wt_hbm, out_hbm,
          drv_idx_v, wk_idx_v, tm_idx_v, dist_v,
          drv_rows_v, wk_tab_v, tm_tab_v, rest_v, sem):
    wid = lax.axis_index("s") * NC + lax.axis_index("c")
    base = wid * B_W

    pltpu.sync_copy(drv_idx_hbm.at[pl.ds(base, B_W)], drv_idx_v)
    g1 = pltpu.async_copy(wd_hbm.at[drv_idx_v], drv_rows_v, sem)
    pltpu.sync_copy(wk_idx_hbm.at[pl.ds(base, B_W)], wk_idx_v)
    pltpu.sync_copy(tm_idx_hbm.at[pl.ds(base, B_W)], tm_idx_v)
    pltpu.sync_copy(dist_hbm.at[pl.ds(base, B_W)], dist_v)
    pltpu.sync_copy(wk_hbm, wk_tab_v)
    pltpu.sync_copy(wt_hbm, tm_tab_v)

    iota = lax.iota(jnp.int32, L)

    def chunk(i, carry):
        rows = i * L + iota
        wk16 = plsc.load_gather(wk_idx_v, [rows]) * WK_STRIDE
        for c in range(D_WK):
            val = plsc.load_gather(wk_tab_v, [wk16 + c])
            plsc.store_scatter(rest_v, [rows, iota * 0 + c], val)
        tm16 = plsc.load_gather(tm_idx_v, [rows]) * D_TM
        for c in range(D_TM):
            val = plsc.load_gather(tm_tab_v, [tm16 + c])
            plsc.store_scatter(rest_v, [rows, iota * 0 + (D_WK + c)], val)
        d16 = plsc.load_gather(dist_v, [rows])
        plsc.store_scatter(rest_v, [rows, iota * 0 + (D_REST - 1)], d16)
        return carry

    lax.fori_loop(0, CHUNKS, chunk, 0)

    rows = out_hbm.at[pl.ds(base, B_W)]
    g1.wait()
    pltpu.sync_copy(drv_rows_v, rows.at[:, pl.ds(0, D_DRV)])
    pltpu.sync_copy(rest_v, rows.at[:, pl.ds(D_DRV, D_REST)])


@jax.jit
def _run(drv_idx, wk_idx, tm_idx, dist, wd, wk, wt):
    mesh = plsc.VectorSubcoreMesh(core_axis_name="c", subcore_axis_name="s")
    f = pl.kernel(
        _body, mesh=mesh,
        compiler_params=pltpu.CompilerParams(
            needs_layout_passes=False, use_tc_tiling_on_sc=False),
        out_type=jax.ShapeDtypeStruct((N, D_OUT), jnp.float32),
        scratch_types=[
            pltpu.VMEM((B_W,), jnp.int32),
            pltpu.VMEM((B_W,), jnp.int32),
            pltpu.VMEM((B_W,), jnp.int32),
            pltpu.VMEM((B_W,), jnp.float32),
            pltpu.VMEM((B_W, D_DRV), jnp.float32),
            pltpu.VMEM((8 * WK_STRIDE,), jnp.float32),
            pltpu.VMEM((1440 * D_TM,), jnp.float32),
            pltpu.VMEM((B_W, D_REST), jnp.float32),
            pltpu.SemaphoreType.DMA,
        ],
    )
    return f(drv_idx, wk_idx, tm_idx, dist, wd, wk, wt)


def kernel(driverID, weekID, timeID, dist, W_driver, W_week, W_time):
    drv_idx = driverID.astype(jnp.int32).reshape(-1)
    wk_idx = weekID.astype(jnp.int32).reshape(-1)
    tm_idx = timeID.astype(jnp.int32).reshape(-1)
    wk_pad = jnp.zeros((8, WK_STRIDE), jnp.float32).at[:7, :D_WK].set(W_week)
    return _run(drv_idx, wk_idx, tm_idx, dist.reshape(-1),
                W_driver, wk_pad.reshape(-1), W_time.reshape(-1))
